# baseline (speedup 1.0000x reference)
import math

import jax
import jax.numpy as jnp
from jax import lax
from jax.experimental import pallas as pl
from jax.experimental.pallas import tpu as pltpu
from jax.experimental.pallas import tpu_sc as plsc

D = 1024
N = 16384
NW = 32
B_PER_W = N // NW
CB = 32
NCH = B_PER_W // CB
SCALE = math.sqrt(float(D))
LANES = 16


def _scale_chunk(buf):
    def row(r, _):
        def col(v, _):
            sl = pl.ds(v * LANES, LANES)
            buf[r, sl] = buf[r, sl] * SCALE
            return 0
        return lax.fori_loop(0, D // LANES, col, 0)
    lax.fori_loop(0, CB, row, 0)


def _body(ids_hbm, table_hbm, out_hbm, idx_v, buf0, buf1, sem0, sem1):
    wid = lax.axis_index("s") * 2 + lax.axis_index("c")
    base = wid * B_PER_W
    pltpu.sync_copy(ids_hbm.at[wid], idx_v)

    bufs = (buf0, buf1)
    sems = (sem0, sem1)
    handles = [None] * NCH
    handles[0] = pltpu.async_copy(table_hbm.at[idx_v.at[0]], buf0, sem0)
    if NCH > 1:
        handles[1] = pltpu.async_copy(table_hbm.at[idx_v.at[1]], buf1, sem1)
    for c in range(NCH):
        k = c % 2
        handles[c].wait()
        _scale_chunk(bufs[k])
        pltpu.sync_copy(bufs[k], out_hbm.at[pl.ds(base + c * CB, CB)])
        if c + 2 < NCH:
            handles[c + 2] = pltpu.async_copy(
                table_hbm.at[idx_v.at[c + 2]], bufs[k], sems[k])


def kernel(input_ids, token_emb):
    ids = input_ids.reshape(-1).astype(jnp.int32).reshape(NW, NCH, CB)
    mesh = plsc.VectorSubcoreMesh(core_axis_name="c", subcore_axis_name="s")
    out = pl.kernel(
        _body,
        out_type=jax.ShapeDtypeStruct((N, D), jnp.float32),
        mesh=mesh,
        scratch_types=[
            pltpu.VMEM((NCH, CB), jnp.int32),
            pltpu.VMEM((CB, D), jnp.float32),
            pltpu.VMEM((CB, D), jnp.float32),
            pltpu.SemaphoreType.DMA,
            pltpu.SemaphoreType.DMA,
        ],
    )(ids, token_emb)
    return out.reshape(input_ids.shape[0], input_ids.shape[1], D)


# speedup vs baseline: 2.5550x; 2.5550x over previous
import math

import jax
import jax.numpy as jnp
from jax import lax
from jax.experimental import pallas as pl
from jax.experimental.pallas import tpu as pltpu
from jax.experimental.pallas import tpu_sc as plsc

D = 1024
N = 16384
NW = 32
B_PER_W = N // NW
CB = 32
NCH = B_PER_W // CB
NB = 3
SCALE = math.sqrt(float(D))
LANES = 16
GROUPS = D // LANES


def _scale_chunk(buf):
    def row(r, _):
        for v in range(GROUPS):
            sl = pl.ds(v * LANES, LANES)
            buf[r, sl] = buf[r, sl] * SCALE
        return 0
    lax.fori_loop(0, CB, row, 0, unroll=False)


def _body(ids_hbm, table_hbm, out_hbm, idx_v,
          buf0, buf1, buf2, sg0, sg1, sg2, so0, so1, so2):
    wid = lax.axis_index("s") * 2 + lax.axis_index("c")
    base = wid * B_PER_W
    pltpu.sync_copy(ids_hbm.at[wid], idx_v)

    bufs = (buf0, buf1, buf2)
    sems_g = (sg0, sg1, sg2)
    sems_o = (so0, so1, so2)

    def gather(c):
        k = c % NB
        return pltpu.async_copy(table_hbm.at[idx_v.at[c]], bufs[k], sems_g[k])

    def scatter(c):
        k = c % NB
        return pltpu.async_copy(
            bufs[k], out_hbm.at[pl.ds(base + c * CB, CB)], sems_o[k])

    hg = [None] * NCH
    ho = [None] * NCH
    hg[0] = gather(0)
    hg[1] = gather(1)
    for c in range(NCH):
        nxt = c + NB - 1
        if nxt < NCH:
            if nxt >= NB:
                ho[nxt - NB].wait()
            hg[nxt] = gather(nxt)
        hg[c].wait()
        _scale_chunk(bufs[c % NB])
        ho[c] = scatter(c)
    for c in range(NCH - NB, NCH):
        if c >= 0 and ho[c] is not None:
            ho[c].wait()


def kernel(input_ids, token_emb):
    ids = input_ids.reshape(-1).astype(jnp.int32).reshape(NW, NCH, CB)
    mesh = plsc.VectorSubcoreMesh(core_axis_name="c", subcore_axis_name="s")
    out = pl.kernel(
        _body,
        out_type=jax.ShapeDtypeStruct((N, D), jnp.float32),
        mesh=mesh,
        scratch_types=[
            pltpu.VMEM((NCH, CB), jnp.int32),
            pltpu.VMEM((CB, D), jnp.float32),
            pltpu.VMEM((CB, D), jnp.float32),
            pltpu.VMEM((CB, D), jnp.float32),
            pltpu.SemaphoreType.DMA,
            pltpu.SemaphoreType.DMA,
            pltpu.SemaphoreType.DMA,
            pltpu.SemaphoreType.DMA,
            pltpu.SemaphoreType.DMA,
            pltpu.SemaphoreType.DMA,
        ],
    )(ids, token_emb)
    return out.reshape(input_ids.shape[0], input_ids.shape[1], D)


# speedup vs baseline: 2.6552x; 1.0392x over previous
import math

import jax
import jax.numpy as jnp
from jax import lax
from jax.experimental import pallas as pl
from jax.experimental.pallas import tpu as pltpu
from jax.experimental.pallas import tpu_sc as plsc

D = 1024
N = 16384
NW = 32
B_PER_W = N // NW
CB = 32
NCH = B_PER_W // CB
NB = 3
SCALE = math.sqrt(float(D))
LANES = 16
GROUPS = D // LANES


SUB = 4
PR = CB // SUB


def _scale_piece(buf, h):
    def row(r, _):
        for v in range(GROUPS):
            sl = pl.ds(v * LANES, LANES)
            buf[r, sl] = buf[r, sl] * SCALE
        return 0
    lax.fori_loop(h * PR, (h + 1) * PR, row, 0, unroll=False)


def _body(ids_hbm, table_hbm, out_hbm, idx_v,
          buf0, buf1, buf2, sg0, sg1, sg2, so0, so1, so2):
    wid = lax.axis_index("s") * 2 + lax.axis_index("c")
    base = wid * B_PER_W
    pltpu.sync_copy(ids_hbm.at[wid], idx_v)

    bufs = (buf0, buf1, buf2)
    sems_g = (sg0, sg1, sg2)
    sems_o = (so0, so1, so2)

    def gather(c):
        k = c % NB
        return pltpu.async_copy(table_hbm.at[idx_v.at[c]], bufs[k], sems_g[k])

    def scatter_piece(c, h):
        k = c % NB
        rows = pl.ds(h * PR, PR)
        return pltpu.async_copy(
            bufs[k].at[rows],
            out_hbm.at[pl.ds(base + c * CB + h * PR, PR)], sems_o[k])

    hg = [None] * NCH
    ho = [[None] * SUB for _ in range(NCH)]
    hg[0] = gather(0)
    hg[1] = gather(1)
    for c in range(NCH):
        nxt = c + NB - 1
        if nxt < NCH:
            if nxt >= NB:
                for h in range(SUB):
                    ho[nxt - NB][h].wait()
            hg[nxt] = gather(nxt)
        hg[c].wait()
        k = c % NB
        for h in range(SUB):
            _scale_piece(bufs[k], h)
            ho[c][h] = scatter_piece(c, h)
    for c in range(NCH - NB, NCH):
        if c >= 0:
            for h in range(SUB):
                if ho[c][h] is not None:
                    ho[c][h].wait()


def kernel(input_ids, token_emb):
    ids = input_ids.reshape(-1).astype(jnp.int32).reshape(NW, NCH, CB)
    mesh = plsc.VectorSubcoreMesh(core_axis_name="c", subcore_axis_name="s")
    out = pl.kernel(
        _body,
        out_type=jax.ShapeDtypeStruct((N, D), jnp.float32),
        mesh=mesh,
        scratch_types=[
            pltpu.VMEM((NCH, CB), jnp.int32),
            pltpu.VMEM((CB, D), jnp.float32),
            pltpu.VMEM((CB, D), jnp.float32),
            pltpu.VMEM((CB, D), jnp.float32),
            pltpu.SemaphoreType.DMA,
            pltpu.SemaphoreType.DMA,
            pltpu.SemaphoreType.DMA,
            pltpu.SemaphoreType.DMA,
            pltpu.SemaphoreType.DMA,
            pltpu.SemaphoreType.DMA,
        ],
    )(ids, token_emb)
    return out.reshape(input_ids.shape[0], input_ids.shape[1], D)
